# baseline (speedup 1.0000x reference)
import jax
import jax.numpy as jnp
from jax import lax
from jax.experimental import pallas as pl
from jax.experimental.pallas import tpu as pltpu
from jax.experimental.pallas import tpu_sc as plsc

V = 100000
D = 300
DP = 384
N_TOK = 51200

NW = 32
PER_W = N_TOK // NW
CHUNK = 80
NCHUNK = PER_W // CHUNK

BV = 1000
BT = 512


def _pad_body(gt_ref, ft_ref, outg_ref, outf_ref):
    z = jnp.zeros((BV, DP - D), dtype=jnp.float32)
    outg_ref[...] = jnp.concatenate([gt_ref[...], z], axis=1)
    outf_ref[...] = jnp.concatenate([ft_ref[...], z], axis=1)


def _tc_pad(gt, ft):
    grid = (V // BV,)
    return pl.pallas_call(
        _pad_body,
        grid=grid,
        in_specs=[
            pl.BlockSpec((BV, D), lambda i: (i, 0)),
            pl.BlockSpec((BV, D), lambda i: (i, 0)),
        ],
        out_specs=[
            pl.BlockSpec((BV, DP), lambda i: (i, 0)),
            pl.BlockSpec((BV, DP), lambda i: (i, 0)),
        ],
        out_shape=[
            jax.ShapeDtypeStruct((V, DP), jnp.float32),
            jax.ShapeDtypeStruct((V, DP), jnp.float32),
        ],
    )(gt, ft)


def _gather_body(gt_hbm, ft_hbm, idx_hbm, outg_hbm, outf_hbm,
                 idx_v, bufg, buff, semg, semf):
    wid = lax.axis_index("s") * 2 + lax.axis_index("c")
    base = wid * PER_W
    pltpu.sync_copy(idx_hbm.at[pl.ds(base, PER_W)], idx_v)

    def step(c, carry):
        off = pl.multiple_of(c * CHUNK, 8)
        idx_c = idx_v.at[pl.ds(off, CHUNK)]
        cg = pltpu.async_copy(gt_hbm.at[idx_c], bufg, semg)
        cf = pltpu.async_copy(ft_hbm.at[idx_c], buff, semf)
        cg.wait()
        pltpu.sync_copy(bufg, outg_hbm.at[pl.ds(base + off, CHUNK)])
        cf.wait()
        pltpu.sync_copy(buff, outf_hbm.at[pl.ds(base + off, CHUNK)])
        return carry

    lax.fori_loop(0, NCHUNK, step, 0)


def _sc_gather(glove_pad, fasttext_pad, idx):
    mesh = plsc.VectorSubcoreMesh(core_axis_name="c", subcore_axis_name="s")
    f = pl.kernel(
        _gather_body,
        out_type=(
            jax.ShapeDtypeStruct((N_TOK, DP), jnp.float32),
            jax.ShapeDtypeStruct((N_TOK, DP), jnp.float32),
        ),
        mesh=mesh,
        scratch_types=[
            pltpu.VMEM((PER_W,), jnp.int32),
            pltpu.VMEM((CHUNK, DP), jnp.float32),
            pltpu.VMEM((CHUNK, DP), jnp.float32),
            pltpu.SemaphoreType.DMA,
            pltpu.SemaphoreType.DMA,
        ],
    )
    return f(glove_pad, fasttext_pad, idx)


def _combine_body(g_ref, f_ref, wg_ref, bg_ref, wf_ref, bf_ref,
                  wa1_ref, wa2_ref, out_ref):
    g = g_ref[:, :D]
    f = f_ref[:, :D]
    dn = (((1,), (1,)), ((), ()))
    g_out = lax.dot_general(g, wg_ref[...], dn,
                            preferred_element_type=jnp.float32) + bg_ref[...]
    f_out = lax.dot_general(f, wf_ref[...], dn,
                            preferred_element_type=jnp.float32) + bf_ref[...]
    wvec = lax.dot_general(wa2_ref[...], wa1_ref[...], (((1,), (0,)), ((), ())),
                           preferred_element_type=jnp.float32)
    diff = jnp.sum((g_out - f_out) * wvec, axis=1, keepdims=True)
    s = 1.0 / (1.0 + jnp.exp(-diff))
    out_ref[...] = jnp.maximum(s * g_out + (1.0 - s) * f_out, 0.0)


def _tc_combine(g_emb, f_emb, Wg, bg, Wf, bf, Wa1, Wa2):
    grid = (N_TOK // BT,)
    return pl.pallas_call(
        _combine_body,
        grid=grid,
        in_specs=[
            pl.BlockSpec((BT, DP), lambda i: (i, 0)),
            pl.BlockSpec((BT, DP), lambda i: (i, 0)),
            pl.BlockSpec((D, D), lambda i: (0, 0)),
            pl.BlockSpec((1, D), lambda i: (0, 0)),
            pl.BlockSpec((D, D), lambda i: (0, 0)),
            pl.BlockSpec((1, D), lambda i: (0, 0)),
            pl.BlockSpec((10, D), lambda i: (0, 0)),
            pl.BlockSpec((1, 10), lambda i: (0, 0)),
        ],
        out_specs=pl.BlockSpec((BT, D), lambda i: (i, 0)),
        out_shape=jax.ShapeDtypeStruct((N_TOK, D), jnp.float32),
    )(g_emb, f_emb, Wg, bg, Wf, bf, Wa1, Wa2)


def kernel(word, glove_table, fasttext_table, Wg, bg, Wf, bf, Wa1, ba1, Wa2, ba2):
    B, L = word.shape
    idx = word.reshape(-1).astype(jnp.int32)
    gtp, ftp = _tc_pad(glove_table, fasttext_table)
    g_emb, f_emb = _sc_gather(gtp, ftp, idx)
    out = _tc_combine(g_emb, f_emb, Wg, bg.reshape(1, D), Wf, bf.reshape(1, D),
                      Wa1, Wa2)
    return out.reshape(B, L, D)


# speedup vs baseline: 1.1231x; 1.1231x over previous
import jax
import jax.numpy as jnp
from jax import lax
from jax.experimental import pallas as pl
from jax.experimental.pallas import tpu as pltpu
from jax.experimental.pallas import tpu_sc as plsc

V = 100000
D = 300
DP = 384
N_TOK = 51200

NW = 32
PER_W = N_TOK // NW
CHUNK = 80
NCHUNK = PER_W // CHUNK

BT = 512


def _gather_body(gt_hbm, ft_hbm, gtail_hbm, ftail_hbm, idx_hbm,
                 outg_hbm, outf_hbm, idx_v, bufg, buff, semg, semf):
    wid = lax.axis_index("s") * 2 + lax.axis_index("c")
    base = wid * PER_W
    pltpu.sync_copy(idx_hbm.at[pl.ds(base, PER_W)], idx_v)

    def step(c, carry):
        off = pl.multiple_of(c * CHUNK, 8)
        idx_c = idx_v.at[pl.ds(off, CHUNK)]
        cg = pltpu.async_copy(gt_hbm.at[:, pl.ds(0, 256)].at[idx_c],
                              bufg.at[:, pl.ds(0, 256)], semg)
        cg2 = pltpu.async_copy(gtail_hbm.at[idx_c],
                               bufg.at[:, pl.ds(256, 128)], semg)
        cf = pltpu.async_copy(ft_hbm.at[:, pl.ds(0, 256)].at[idx_c],
                              buff.at[:, pl.ds(0, 256)], semf)
        cf2 = pltpu.async_copy(ftail_hbm.at[idx_c],
                               buff.at[:, pl.ds(256, 128)], semf)
        cg.wait()
        cg2.wait()
        pltpu.sync_copy(bufg, outg_hbm.at[pl.ds(base + off, CHUNK)])
        cf.wait()
        cf2.wait()
        pltpu.sync_copy(buff, outf_hbm.at[pl.ds(base + off, CHUNK)])
        return carry

    lax.fori_loop(0, NCHUNK, step, 0)


def _sc_gather(gt, ft, gtail, ftail, idx):
    mesh = plsc.VectorSubcoreMesh(core_axis_name="c", subcore_axis_name="s")
    f = pl.kernel(
        _gather_body,
        out_type=(
            jax.ShapeDtypeStruct((N_TOK, DP), jnp.float32),
            jax.ShapeDtypeStruct((N_TOK, DP), jnp.float32),
        ),
        mesh=mesh,
        scratch_types=[
            pltpu.VMEM((PER_W,), jnp.int32),
            pltpu.VMEM((CHUNK, DP), jnp.float32),
            pltpu.VMEM((CHUNK, DP), jnp.float32),
            pltpu.SemaphoreType.DMA,
            pltpu.SemaphoreType.DMA,
        ],
    )
    return f(gt, ft, gtail, ftail, idx)


def _combine_body(g_ref, f_ref, wg_ref, bg_ref, wf_ref, bf_ref,
                  wa1_ref, wa2_ref, out_ref):
    gh, gt = g_ref[:, :256], g_ref[:, 340:DP]
    fh, ft = f_ref[:, :256], f_ref[:, 340:DP]
    wg = wg_ref[...]
    wf = wf_ref[...]
    dn = (((1,), (1,)), ((), ()))
    g_out = (lax.dot_general(gh, wg[:, :256], dn,
                             preferred_element_type=jnp.float32)
             + lax.dot_general(gt, wg[:, 256:], dn,
                               preferred_element_type=jnp.float32)
             + bg_ref[...])
    f_out = (lax.dot_general(fh, wf[:, :256], dn,
                             preferred_element_type=jnp.float32)
             + lax.dot_general(ft, wf[:, 256:], dn,
                               preferred_element_type=jnp.float32)
             + bf_ref[...])
    wvec = lax.dot_general(wa2_ref[...], wa1_ref[...], (((1,), (0,)), ((), ())),
                           preferred_element_type=jnp.float32)
    diff = jnp.sum((g_out - f_out) * wvec, axis=1, keepdims=True)
    s = 1.0 / (1.0 + jnp.exp(-diff))
    out_ref[...] = jnp.maximum(s * g_out + (1.0 - s) * f_out, 0.0)


def _tc_combine(g_emb, f_emb, Wg, bg, Wf, bf, Wa1, Wa2):
    grid = (N_TOK // BT,)
    return pl.pallas_call(
        _combine_body,
        grid=grid,
        in_specs=[
            pl.BlockSpec((BT, DP), lambda i: (i, 0)),
            pl.BlockSpec((BT, DP), lambda i: (i, 0)),
            pl.BlockSpec((D, D), lambda i: (0, 0)),
            pl.BlockSpec((1, D), lambda i: (0, 0)),
            pl.BlockSpec((D, D), lambda i: (0, 0)),
            pl.BlockSpec((1, D), lambda i: (0, 0)),
            pl.BlockSpec((10, D), lambda i: (0, 0)),
            pl.BlockSpec((1, 10), lambda i: (0, 0)),
        ],
        out_specs=pl.BlockSpec((BT, D), lambda i: (i, 0)),
        out_shape=jax.ShapeDtypeStruct((N_TOK, D), jnp.float32),
    )(g_emb, f_emb, Wg, bg, Wf, bf, Wa1, Wa2)


def kernel(word, glove_table, fasttext_table, Wg, bg, Wf, bf, Wa1, ba1, Wa2, ba2):
    B, L = word.shape
    idx = word.reshape(-1).astype(jnp.int32)
    gtail = lax.slice(glove_table, (0, 172), (V, D))
    ftail = lax.slice(fasttext_table, (0, 172), (V, D))
    g_emb, f_emb = _sc_gather(glove_table, fasttext_table, gtail, ftail, idx)
    out = _tc_combine(g_emb, f_emb, Wg, bg.reshape(1, D), Wf, bf.reshape(1, D),
                      Wa1, Wa2)
    return out.reshape(B, L, D)


# speedup vs baseline: 1.1684x; 1.0403x over previous
import jax
import jax.numpy as jnp
from jax import lax
from jax.experimental import pallas as pl
from jax.experimental.pallas import tpu as pltpu
from jax.experimental.pallas import tpu_sc as plsc

V = 100000
D = 300
DP = 384
N_TOK = 51200

NW = 32
PER_W = N_TOK // NW
CHUNK = 80
NCHUNK = PER_W // CHUNK

BT = 400
BB = 8


def _gather_body(gt_hbm, ft_hbm, gtail_hbm, ftail_hbm, idx_hbm,
                 outg_hbm, outf_hbm, idx_v, bufg, buff, semg, semf):
    wid = lax.axis_index("s") * 2 + lax.axis_index("c")
    base = wid * PER_W
    pltpu.sync_copy(idx_hbm.at[pl.ds(base, PER_W)], idx_v)

    def step(c, carry):
        off = pl.multiple_of(c * CHUNK, 8)
        idx_c = idx_v.at[pl.ds(off, CHUNK)]
        cg = pltpu.async_copy(gt_hbm.at[:, pl.ds(0, 256)].at[idx_c],
                              bufg.at[:, pl.ds(0, 256)], semg)
        cg2 = pltpu.async_copy(gtail_hbm.at[idx_c],
                               bufg.at[:, pl.ds(256, 128)], semg)
        cf = pltpu.async_copy(ft_hbm.at[:, pl.ds(0, 256)].at[idx_c],
                              buff.at[:, pl.ds(0, 256)], semf)
        cf2 = pltpu.async_copy(ftail_hbm.at[idx_c],
                               buff.at[:, pl.ds(256, 128)], semf)
        cg.wait()
        cg2.wait()
        pltpu.sync_copy(bufg, outg_hbm.at[pl.ds(base + off, CHUNK)])
        cf.wait()
        cf2.wait()
        pltpu.sync_copy(buff, outf_hbm.at[pl.ds(base + off, CHUNK)])
        return carry

    lax.fori_loop(0, NCHUNK, step, 0)


def _sc_gather(gt, ft, gtail, ftail, idx):
    mesh = plsc.VectorSubcoreMesh(core_axis_name="c", subcore_axis_name="s")
    f = pl.kernel(
        _gather_body,
        out_type=(
            jax.ShapeDtypeStruct((N_TOK, DP), jnp.float32),
            jax.ShapeDtypeStruct((N_TOK, DP), jnp.float32),
        ),
        mesh=mesh,
        scratch_types=[
            pltpu.VMEM((PER_W,), jnp.int32),
            pltpu.VMEM((CHUNK, DP), jnp.float32),
            pltpu.VMEM((CHUNK, DP), jnp.float32),
            pltpu.SemaphoreType.DMA,
            pltpu.SemaphoreType.DMA,
        ],
    )
    return f(gt, ft, gtail, ftail, idx)


def _combine_body(g_ref, f_ref, wg_ref, bg_ref, wf_ref, bf_ref,
                  wa1_ref, wa2_ref, out_ref):
    gh, gt = g_ref[:, :256], g_ref[:, 340:DP]
    fh, ft = f_ref[:, :256], f_ref[:, 340:DP]
    wg = wg_ref[...]
    wf = wf_ref[...]
    dn = (((1,), (1,)), ((), ()))
    g_out = (lax.dot_general(gh, wg[:, :256], dn,
                             preferred_element_type=jnp.float32)
             + lax.dot_general(gt, wg[:, 256:], dn,
                               preferred_element_type=jnp.float32)
             + bg_ref[...])
    f_out = (lax.dot_general(fh, wf[:, :256], dn,
                             preferred_element_type=jnp.float32)
             + lax.dot_general(ft, wf[:, 256:], dn,
                               preferred_element_type=jnp.float32)
             + bf_ref[...])
    wvec = lax.dot_general(wa2_ref[...], wa1_ref[...], (((1,), (0,)), ((), ())),
                           preferred_element_type=jnp.float32)
    diff = jnp.sum((g_out - f_out) * wvec, axis=1, keepdims=True)
    s = 1.0 / (1.0 + jnp.exp(-diff))
    res = jnp.maximum(s * g_out + (1.0 - s) * f_out, 0.0)
    out_ref[...] = res.reshape(out_ref.shape)


def _tc_combine(g_emb, f_emb, Wg, bg, Wf, bf, Wa1, Wa2, B, L):
    grid = (N_TOK // BT,)
    return pl.pallas_call(
        _combine_body,
        grid=grid,
        in_specs=[
            pl.BlockSpec((BT, DP), lambda i: (i, 0)),
            pl.BlockSpec((BT, DP), lambda i: (i, 0)),
            pl.BlockSpec((D, D), lambda i: (0, 0)),
            pl.BlockSpec((1, D), lambda i: (0, 0)),
            pl.BlockSpec((D, D), lambda i: (0, 0)),
            pl.BlockSpec((1, D), lambda i: (0, 0)),
            pl.BlockSpec((10, D), lambda i: (0, 0)),
            pl.BlockSpec((1, 10), lambda i: (0, 0)),
        ],
        out_specs=pl.BlockSpec((BB, L, D), lambda i: (i, 0, 0)),
        out_shape=jax.ShapeDtypeStruct((B, L, D), jnp.float32),
    )(g_emb, f_emb, Wg, bg, Wf, bf, Wa1, Wa2)


def kernel(word, glove_table, fasttext_table, Wg, bg, Wf, bf, Wa1, ba1, Wa2, ba2):
    B, L = word.shape
    idx = word.reshape(-1).astype(jnp.int32)
    gtail = lax.slice(glove_table, (0, 172), (V, D))
    ftail = lax.slice(fasttext_table, (0, 172), (V, D))
    g_emb, f_emb = _sc_gather(glove_table, fasttext_table, gtail, ftail, idx)
    return _tc_combine(g_emb, f_emb, Wg, bg.reshape(1, D), Wf, bf.reshape(1, D),
                       Wa1, Wa2, B, L)


# speedup vs baseline: 1.2600x; 1.0784x over previous
import jax
import jax.numpy as jnp
from jax import lax
from jax.experimental import pallas as pl
from jax.experimental.pallas import tpu as pltpu
from jax.experimental.pallas import tpu_sc as plsc

V = 100000
D = 300
DP = 384
N_TOK = 51200

NW = 32
PER_W = N_TOK // NW
CHUNK = 80
NCHUNK = PER_W // CHUNK

BT = 800
BB = 16


def _gather_body(gt_hbm, gtail_hbm, idx_hbm, outg_hbm, idx_v, bufg, semg):
    wid = lax.axis_index("s") * 2 + lax.axis_index("c")
    base = wid * PER_W
    pltpu.sync_copy(idx_hbm.at[pl.ds(base, PER_W)], idx_v)

    def step(c, carry):
        off = pl.multiple_of(c * CHUNK, 8)
        idx_c = idx_v.at[pl.ds(off, CHUNK)]
        cg = pltpu.async_copy(gt_hbm.at[:, pl.ds(0, 256)].at[idx_c],
                              bufg.at[:, pl.ds(0, 256)], semg)
        cg2 = pltpu.async_copy(gtail_hbm.at[idx_c],
                               bufg.at[:, pl.ds(256, 128)], semg)
        cg.wait()
        cg2.wait()
        pltpu.sync_copy(bufg, outg_hbm.at[pl.ds(base + off, CHUNK)])
        return carry

    lax.fori_loop(0, NCHUNK, step, 0)


def _sc_gather_one(table, tail, idx):
    mesh = plsc.VectorSubcoreMesh(core_axis_name="c", subcore_axis_name="s")
    f = pl.kernel(
        _gather_body,
        out_type=jax.ShapeDtypeStruct((N_TOK, DP), jnp.float32),
        mesh=mesh,
        scratch_types=[
            pltpu.VMEM((PER_W,), jnp.int32),
            pltpu.VMEM((CHUNK, DP), jnp.float32),
            pltpu.SemaphoreType.DMA,
        ],
    )
    return f(table, tail, idx)


def _combine_body(g_ref, f_ref, wg_ref, bg_ref, wf_ref, bf_ref,
                  wa1_ref, wa2_ref, out_ref):
    gh, gt = g_ref[:, :256], g_ref[:, 340:DP]
    fh, ft = f_ref[:, :256], f_ref[:, 340:DP]
    wg = wg_ref[...]
    wf = wf_ref[...]
    dn = (((1,), (1,)), ((), ()))
    g_out = (lax.dot_general(gh, wg[:, :256], dn,
                             preferred_element_type=jnp.float32)
             + lax.dot_general(gt, wg[:, 256:], dn,
                               preferred_element_type=jnp.float32)
             + bg_ref[...])
    f_out = (lax.dot_general(fh, wf[:, :256], dn,
                             preferred_element_type=jnp.float32)
             + lax.dot_general(ft, wf[:, 256:], dn,
                               preferred_element_type=jnp.float32)
             + bf_ref[...])
    wvec = lax.dot_general(wa2_ref[...], wa1_ref[...], (((1,), (0,)), ((), ())),
                           preferred_element_type=jnp.float32)
    diff = jnp.sum((g_out - f_out) * wvec, axis=1, keepdims=True)
    s = 1.0 / (1.0 + jnp.exp(-diff))
    res = jnp.maximum(s * g_out + (1.0 - s) * f_out, 0.0)
    out_ref[...] = res.reshape(out_ref.shape)


def _tc_combine(g_emb, f_emb, Wg, bg, Wf, bf, Wa1, Wa2, B, L):
    grid = (N_TOK // BT,)
    return pl.pallas_call(
        _combine_body,
        grid=grid,
        in_specs=[
            pl.BlockSpec((BT, DP), lambda i: (i, 0)),
            pl.BlockSpec((BT, DP), lambda i: (i, 0)),
            pl.BlockSpec((D, D), lambda i: (0, 0)),
            pl.BlockSpec((1, D), lambda i: (0, 0)),
            pl.BlockSpec((D, D), lambda i: (0, 0)),
            pl.BlockSpec((1, D), lambda i: (0, 0)),
            pl.BlockSpec((10, D), lambda i: (0, 0)),
            pl.BlockSpec((1, 10), lambda i: (0, 0)),
        ],
        out_specs=pl.BlockSpec((BB, L, D), lambda i: (i, 0, 0)),
        out_shape=jax.ShapeDtypeStruct((B, L, D), jnp.float32),
    )(g_emb, f_emb, Wg, bg, Wf, bf, Wa1, Wa2)


def kernel(word, glove_table, fasttext_table, Wg, bg, Wf, bf, Wa1, ba1, Wa2, ba2):
    B, L = word.shape
    idx = word.reshape(-1).astype(jnp.int32)
    gtail = lax.slice(glove_table, (0, 172), (V, D))
    ftail = lax.slice(fasttext_table, (0, 172), (V, D))
    g_emb = _sc_gather_one(glove_table, gtail, idx)
    f_emb = _sc_gather_one(fasttext_table, ftail, idx)
    return _tc_combine(g_emb, f_emb, Wg, bg.reshape(1, D), Wf, bf.reshape(1, D),
                       Wa1, Wa2, B, L)
